# baseline (speedup 1.0000x reference)
import functools
import math

import jax
import jax.numpy as jnp
from jax import lax
from jax.experimental import pallas as pl
from jax.experimental.pallas import tpu as pltpu
from jax.experimental.pallas import tpu_sc as plsc

EMB = 64
SCALE = math.sqrt(EMB)

NUM_WORKERS = 32
CHUNK = 256
ROWS_A = 4000
SEQ_C = 8


def _widen_table(table):
    V, D = table.shape

    def body(x_ref, o_ref):
        o_ref[:, :D] = x_ref[...]
        o_ref[:, D:] = jnp.zeros((ROWS_A, D), jnp.float32)

    return pl.pallas_call(
        body,
        grid=(V // ROWS_A,),
        in_specs=[pl.BlockSpec((ROWS_A, D), lambda i: (i, 0))],
        out_specs=pl.BlockSpec((ROWS_A, 2 * D), lambda i: (i, 0)),
        out_shape=jax.ShapeDtypeStruct((V, 2 * D), jnp.float32),
    )(table)


def _finish(mid, Bseq, L):
    def body(x_ref, o_ref):
        y = x_ref[:, :EMB] * SCALE
        o_ref[...] = y.reshape(SEQ_C, L, EMB)

    return pl.pallas_call(
        body,
        grid=(Bseq // SEQ_C,),
        in_specs=[pl.BlockSpec((SEQ_C * L, 2 * EMB), lambda i: (i, 0))],
        out_specs=pl.BlockSpec((SEQ_C, L, EMB), lambda i: (i, 0, 0)),
        out_shape=jax.ShapeDtypeStruct((Bseq, L, EMB), jnp.float32),
    )(mid)


def _make_sc_gather(B, b_per_w, n_chunks):
    mesh = plsc.VectorSubcoreMesh(core_axis_name="c", subcore_axis_name="s")

    @functools.partial(
        pl.kernel,
        mesh=mesh,
        out_type=jax.ShapeDtypeStruct((B, 2 * EMB), jnp.float32),
        scratch_types=[
            pltpu.VMEM((CHUNK,), jnp.int32),
            pltpu.VMEM((CHUNK,), jnp.int32),
            pltpu.VMEM((CHUNK, 2 * EMB), jnp.float32),
            pltpu.VMEM((CHUNK, 2 * EMB), jnp.float32),
            pltpu.SemaphoreType.DMA,
            pltpu.SemaphoreType.DMA,
            pltpu.SemaphoreType.DMA,
            pltpu.SemaphoreType.DMA,
        ],
    )
    def gather_k(tpad_hbm, idx_hbm, mid_hbm,
                 iring0, iring1, rows0, rows1,
                 gsem0, gsem1, ssem0, ssem1):
        iring = (iring0, iring1)
        rows = (rows0, rows1)
        gsem = (gsem0, gsem1)
        ssem = (ssem0, ssem1)
        wid = lax.axis_index("s") * 2 + lax.axis_index("c")
        tbase = wid * b_per_w

        def load_idx(g, b):
            pltpu.sync_copy(idx_hbm.at[pl.ds(tbase + g * CHUNK, CHUNK)],
                            iring[b])

        def gather(g, b):
            pltpu.async_copy(tpad_hbm.at[iring[b]], rows[b], gsem[b])

        def wait_gather(g, b):
            pltpu.make_async_copy(tpad_hbm.at[iring[b]], rows[b],
                                  gsem[b]).wait()

        def scatter(g, b):
            pltpu.async_copy(
                rows[b], mid_hbm.at[pl.ds(tbase + g * CHUNK, CHUNK)],
                ssem[b])

        def wait_scatter(g, b):
            pltpu.make_async_copy(
                rows[b], mid_hbm.at[pl.ds(tbase + g * CHUNK, CHUNK)],
                ssem[b]).wait()

        def step(g, b, first, last):
            other = 1 - b
            if not first:
                wait_scatter(g - 1, other)
            if not last:
                load_idx(g + 1, other)
                gather(g + 1, other)
            wait_gather(g, b)
            scatter(g, b)

        load_idx(0, 0)
        gather(0, 0)
        step(0, 0, first=True, last=False)
        step(1, 1, first=False, last=False)

        def outer(t, carry):
            g = 2 * t
            step(g, 0, first=False, last=False)
            step(g + 1, 1, first=False, last=False)
            return carry

        lax.fori_loop(1, n_chunks // 2 - 1, outer, 0)
        step(n_chunks - 2, 0, first=False, last=False)
        step(n_chunks - 1, 1, first=False, last=True)
        wait_scatter(n_chunks - 1, 1)

    return gather_k


def kernel(token_sequences, table):
    Bseq, L = token_sequences.shape
    B = Bseq * L
    b_per_w = B // NUM_WORKERS
    n_chunks = b_per_w // CHUNK
    idx_flat = token_sequences.reshape(B)
    tpad = _widen_table(table)
    mid = _make_sc_gather(B, b_per_w, n_chunks)(tpad, idx_flat)
    return _finish(mid, Bseq, L)


# speedup vs baseline: 1.2585x; 1.2585x over previous
import functools
import math

import jax
import jax.numpy as jnp
from jax import lax
from jax.experimental import pallas as pl
from jax.experimental.pallas import tpu as pltpu
from jax.experimental.pallas import tpu_sc as plsc

EMB = 64
SCALE = math.sqrt(EMB)

NUM_WORKERS = 32
SEQ_CHUNK = 4
L = 200
CHUNK = SEQ_CHUNK * L
TCOLS = 4096


def _transpose_table(tableT):
    D, V = tableT.shape

    def body(x_ref, o_ref):
        o_ref[...] = x_ref[...].T

    return pl.pallas_call(
        body,
        grid=(pl.cdiv(V, TCOLS),),
        in_specs=[pl.BlockSpec((D, TCOLS), lambda i: (0, i))],
        out_specs=pl.BlockSpec((TCOLS, D), lambda i: (i, 0)),
        out_shape=jax.ShapeDtypeStruct((V, D), jnp.float32),
    )(tableT)


def _make_sc_lookup(Bseq, b_per_w, n_chunks):
    mesh = plsc.VectorSubcoreMesh(core_axis_name="c", subcore_axis_name="s")

    @functools.partial(
        pl.kernel,
        mesh=mesh,
        out_type=jax.ShapeDtypeStruct((Bseq, L, EMB), jnp.float32),
        scratch_types=[
            pltpu.VMEM((CHUNK,), jnp.int32),
            pltpu.VMEM((CHUNK,), jnp.int32),
            pltpu.VMEM((CHUNK, EMB), jnp.float32),
            pltpu.VMEM((CHUNK, EMB), jnp.float32),
            pltpu.SemaphoreType.DMA,
            pltpu.SemaphoreType.DMA,
            pltpu.SemaphoreType.DMA,
            pltpu.SemaphoreType.DMA,
        ],
        compiler_params=pltpu.CompilerParams(use_tc_tiling_on_sc=False),
    )
    def lookup(table_hbm, idx_hbm, out_hbm,
               iring0, iring1, rows0, rows1,
               gsem0, gsem1, ssem0, ssem1):
        iring = (iring0, iring1)
        rows = (rows0, rows1)
        gsem = (gsem0, gsem1)
        ssem = (ssem0, ssem1)
        wid = lax.axis_index("s") * 2 + lax.axis_index("c")
        tbase = wid * b_per_w
        sbase = wid * (b_per_w // L)

        def load_idx(g, b):
            pltpu.sync_copy(idx_hbm.at[pl.ds(tbase + g * CHUNK, CHUNK)],
                            iring[b])

        def gather(g, b):
            pltpu.async_copy(table_hbm.at[iring[b]], rows[b], gsem[b])

        def wait_gather(g, b):
            pltpu.make_async_copy(table_hbm.at[iring[b]], rows[b],
                                  gsem[b]).wait()

        def scatter(g, b):
            for i in range(SEQ_CHUNK):
                pltpu.async_copy(
                    rows[b].at[pl.ds(i * L, L)],
                    out_hbm.at[sbase + g * SEQ_CHUNK + i], ssem[b])

        def wait_scatter(g, b):
            for i in range(SEQ_CHUNK):
                pltpu.make_async_copy(
                    rows[b].at[pl.ds(i * L, L)],
                    out_hbm.at[sbase + g * SEQ_CHUNK + i], ssem[b]).wait()

        def scale(b):
            buf = rows[b]

            @plsc.parallel_loop(0, CHUNK, unroll=4)
            def _(i):
                for j in range(EMB // 16):
                    sl = pl.ds(j * 16, 16)
                    buf[i, sl] = buf[i, sl] * SCALE

        def step(g, b, first, last):
            other = 1 - b
            if not first:
                wait_scatter(g - 1, other)
            if not last:
                load_idx(g + 1, other)
                gather(g + 1, other)
            wait_gather(g, b)
            scale(b)
            scatter(g, b)

        load_idx(0, 0)
        gather(0, 0)
        step(0, 0, first=True, last=False)
        step(1, 1, first=False, last=False)

        def outer(t, carry):
            g = 2 * t
            step(g, 0, first=False, last=False)
            step(g + 1, 1, first=False, last=False)
            return carry

        lax.fori_loop(1, n_chunks // 2 - 1, outer, 0)
        step(n_chunks - 2, 0, first=False, last=False)
        step(n_chunks - 1, 1, first=False, last=True)
        wait_scatter(n_chunks - 1, 1)

    return lookup


def kernel(token_sequences, table):
    Bseq, Lx = token_sequences.shape
    B = Bseq * Lx
    b_per_w = B // NUM_WORKERS
    n_chunks = b_per_w // CHUNK
    idx_flat = token_sequences.reshape(B)
    trow = _transpose_table(table.T)
    return _make_sc_lookup(Bseq, b_per_w, n_chunks)(trow, idx_flat)


# speedup vs baseline: 1.5667x; 1.2449x over previous
import functools
import math

import jax
import jax.numpy as jnp
from jax import lax
from jax.experimental import pallas as pl
from jax.experimental.pallas import tpu as pltpu
from jax.experimental.pallas import tpu_sc as plsc

EMB = 64
SCALE = math.sqrt(EMB)

NUM_WORKERS = 32
L = 200
TCOLS = 4096


def _transpose_widen_table(tableT):
    D, V = tableT.shape

    def body(x_ref, o_ref):
        o_ref[:, :D] = x_ref[...].T
        o_ref[:, D:] = jnp.zeros((TCOLS, D), jnp.float32)

    return pl.pallas_call(
        body,
        grid=(pl.cdiv(V, TCOLS),),
        in_specs=[pl.BlockSpec((D, TCOLS), lambda i: (0, i))],
        out_specs=pl.BlockSpec((TCOLS, 2 * D), lambda i: (i, 0)),
        out_shape=jax.ShapeDtypeStruct((V, 2 * D), jnp.float32),
    )(tableT)


def _make_sc_lookup(Bseq, b_per_w, n_chunks):
    mesh = plsc.VectorSubcoreMesh(core_axis_name="c", subcore_axis_name="s")

    @functools.partial(
        pl.kernel,
        mesh=mesh,
        out_type=jax.ShapeDtypeStruct((Bseq, L, EMB), jnp.float32),
        scratch_types=[
            pltpu.VMEM((L,), jnp.int32),
            pltpu.VMEM((L,), jnp.int32),
            pltpu.VMEM((L, 2 * EMB), jnp.float32),
            pltpu.VMEM((L, 2 * EMB), jnp.float32),
            pltpu.VMEM((L, EMB), jnp.float32),
            pltpu.VMEM((L, EMB), jnp.float32),
            pltpu.SemaphoreType.DMA,
            pltpu.SemaphoreType.DMA,
            pltpu.SemaphoreType.DMA,
            pltpu.SemaphoreType.DMA,
        ],
        compiler_params=pltpu.CompilerParams(use_tc_tiling_on_sc=False),
    )
    def lookup(table_hbm, idx_hbm, out_hbm,
               iring0, iring1, rows0, rows1, obuf0, obuf1,
               gsem0, gsem1, ssem0, ssem1):
        iring = (iring0, iring1)
        rows = (rows0, rows1)
        obuf = (obuf0, obuf1)
        gsem = (gsem0, gsem1)
        ssem = (ssem0, ssem1)
        wid = lax.axis_index("s") * 2 + lax.axis_index("c")
        tbase = wid * b_per_w
        sbase = wid * (b_per_w // L)

        def load_idx(g, b):
            pltpu.sync_copy(idx_hbm.at[pl.ds(tbase + g * L, L)], iring[b])

        def gather(g, b):
            pltpu.async_copy(table_hbm.at[iring[b]], rows[b], gsem[b])

        def wait_gather(g, b):
            pltpu.make_async_copy(table_hbm.at[iring[b]], rows[b],
                                  gsem[b]).wait()

        def scatter(g, b):
            pltpu.async_copy(obuf[b], out_hbm.at[sbase + g], ssem[b])

        def wait_scatter(g, b):
            pltpu.make_async_copy(obuf[b], out_hbm.at[sbase + g],
                                  ssem[b]).wait()

        def pack_scale(b):
            src = rows[b]
            dst = obuf[b]

            @plsc.parallel_loop(0, L, unroll=4)
            def _(i):
                for j in range(EMB // 16):
                    sl = pl.ds(j * 16, 16)
                    dst[i, sl] = src[i, sl] * SCALE

        def step(g, b, first, last):
            other = 1 - b
            if not first:
                wait_scatter(g - 1, other)
            if not last:
                load_idx(g + 1, other)
                gather(g + 1, other)
            wait_gather(g, b)
            pack_scale(b)
            scatter(g, b)

        load_idx(0, 0)
        gather(0, 0)
        step(0, 0, first=True, last=False)
        step(1, 1, first=False, last=False)

        def outer(t, carry):
            g = 2 * t
            step(g, 0, first=False, last=False)
            step(g + 1, 1, first=False, last=False)
            return carry

        lax.fori_loop(1, n_chunks // 2 - 1, outer, 0)
        step(n_chunks - 2, 0, first=False, last=False)
        step(n_chunks - 1, 1, first=False, last=True)
        wait_scatter(n_chunks - 1, 1)

    return lookup


def kernel(token_sequences, table):
    Bseq, Lx = token_sequences.shape
    B = Bseq * Lx
    b_per_w = B // NUM_WORKERS
    n_chunks = b_per_w // L
    idx_flat = token_sequences.reshape(B)
    twide = _transpose_widen_table(table.T)
    return _make_sc_lookup(Bseq, b_per_w, n_chunks)(twide, idx_flat)


# speedup vs baseline: 1.6266x; 1.0382x over previous
import functools
import math

import jax
import jax.numpy as jnp
from jax import lax
from jax.experimental import pallas as pl
from jax.experimental.pallas import tpu as pltpu
from jax.experimental.pallas import tpu_sc as plsc

EMB = 64
SCALE = math.sqrt(EMB)

NUM_WORKERS = 32
L = 200
TCOLS = 4096


def _transpose_widen_table(tableT):
    D, V = tableT.shape

    def body(x_ref, o_ref):
        o_ref[:, :D] = x_ref[...].T

    return pl.pallas_call(
        body,
        grid=(pl.cdiv(V, TCOLS),),
        in_specs=[pl.BlockSpec((D, TCOLS), lambda i: (0, i))],
        out_specs=pl.BlockSpec((TCOLS, 2 * D), lambda i: (i, 0)),
        out_shape=jax.ShapeDtypeStruct((V, 2 * D), jnp.float32),
    )(tableT)


def _make_sc_lookup(Bseq, b_per_w, n_chunks):
    mesh = plsc.VectorSubcoreMesh(core_axis_name="c", subcore_axis_name="s")

    @functools.partial(
        pl.kernel,
        mesh=mesh,
        out_type=jax.ShapeDtypeStruct((Bseq, L, EMB), jnp.float32),
        scratch_types=[
            pltpu.VMEM((b_per_w,), jnp.int32),
            pltpu.VMEM((L, 2 * EMB), jnp.float32),
            pltpu.VMEM((L, 2 * EMB), jnp.float32),
            pltpu.VMEM((L, EMB), jnp.float32),
            pltpu.VMEM((L, EMB), jnp.float32),
            pltpu.SemaphoreType.DMA,
            pltpu.SemaphoreType.DMA,
            pltpu.SemaphoreType.DMA,
            pltpu.SemaphoreType.DMA,
        ],
        compiler_params=pltpu.CompilerParams(use_tc_tiling_on_sc=False),
    )
    def lookup(table_hbm, idx_hbm, out_hbm,
               idx_v, rows0, rows1, obuf0, obuf1,
               gsem0, gsem1, ssem0, ssem1):
        rows = (rows0, rows1)
        obuf = (obuf0, obuf1)
        gsem = (gsem0, gsem1)
        ssem = (ssem0, ssem1)
        wid = lax.axis_index("s") * 2 + lax.axis_index("c")
        tbase = wid * b_per_w
        sbase = wid * (b_per_w // L)

        def gather(g, b):
            pltpu.async_copy(table_hbm.at[idx_v.at[pl.ds(g * L, L)]],
                             rows[b], gsem[b])

        def wait_gather(g, b):
            pltpu.make_async_copy(table_hbm.at[idx_v.at[pl.ds(g * L, L)]],
                                  rows[b], gsem[b]).wait()

        def scatter(g, b):
            pltpu.async_copy(obuf[b], out_hbm.at[sbase + g], ssem[b])

        def wait_scatter(g, b):
            pltpu.make_async_copy(obuf[b], out_hbm.at[sbase + g],
                                  ssem[b]).wait()

        def pack_scale(b):
            src = rows[b]
            dst = obuf[b]

            @plsc.parallel_loop(0, L, unroll=4)
            def _(i):
                for j in range(EMB // 16):
                    sl = pl.ds(j * 16, 16)
                    dst[i, sl] = src[i, sl] * SCALE

        def step(g, b, first, last):
            other = 1 - b
            if not first:
                wait_scatter(g - 1, other)
            if not last:
                gather(g + 1, other)
            wait_gather(g, b)
            pack_scale(b)
            scatter(g, b)

        pltpu.sync_copy(idx_hbm.at[pl.ds(tbase, b_per_w)], idx_v)
        gather(0, 0)
        step(0, 0, first=True, last=False)
        step(1, 1, first=False, last=False)

        def outer(t, carry):
            g = 2 * t
            step(g, 0, first=False, last=False)
            step(g + 1, 1, first=False, last=False)
            return carry

        lax.fori_loop(1, n_chunks // 2 - 1, outer, 0)
        step(n_chunks - 2, 0, first=False, last=False)
        step(n_chunks - 1, 1, first=False, last=True)
        wait_scatter(n_chunks - 1, 1)

    return lookup


def kernel(token_sequences, table):
    Bseq, Lx = token_sequences.shape
    B = Bseq * Lx
    b_per_w = B // NUM_WORKERS
    n_chunks = b_per_w // L
    idx_flat = token_sequences.reshape(B)
    twide = _transpose_widen_table(table.T)
    return _make_sc_lookup(Bseq, b_per_w, n_chunks)(twide, idx_flat)


# speedup vs baseline: 1.7467x; 1.0738x over previous
import functools
import math

import jax
import jax.numpy as jnp
from jax import lax
from jax.experimental import pallas as pl
from jax.experimental.pallas import tpu as pltpu
from jax.experimental.pallas import tpu_sc as plsc

EMB = 64
SCALE = math.sqrt(EMB)

NUM_WORKERS = 32
L = 200
PAIRS = L // 2
TCOLS = 4096
CSEQ = 128


def _transpose_widen_table(tableT):
    D, V = tableT.shape

    def body(x_ref, o_ref):
        o_ref[:, :D] = x_ref[...].T

    return pl.pallas_call(
        body,
        grid=(pl.cdiv(V, TCOLS),),
        in_specs=[pl.BlockSpec((D, TCOLS), lambda i: (0, i))],
        out_specs=pl.BlockSpec((TCOLS, 2 * D), lambda i: (i, 0)),
        out_shape=jax.ShapeDtypeStruct((V, 2 * D), jnp.float32),
    )(tableT)


def _finish_transpose(mid, Bseq):
    def body(x_ref, o_ref):
        x = x_ref[...]
        for l in range(L):
            p, h = divmod(l, 2)
            o_ref[l] = x[:, p, h * EMB:(h + 1) * EMB].T

    return pl.pallas_call(
        body,
        grid=(Bseq // CSEQ,),
        in_specs=[pl.BlockSpec((CSEQ, PAIRS, 2 * EMB), lambda i: (i, 0, 0))],
        out_specs=pl.BlockSpec((L, EMB, CSEQ), lambda i: (0, 0, i)),
        out_shape=jax.ShapeDtypeStruct((L, EMB, Bseq), jnp.float32),
    )(mid)


def _make_sc_lookup(B, b_per_w, n_chunks):
    mesh = plsc.VectorSubcoreMesh(core_axis_name="c", subcore_axis_name="s")

    @functools.partial(
        pl.kernel,
        mesh=mesh,
        out_type=jax.ShapeDtypeStruct((B // L, PAIRS, 2 * EMB),
                                      jnp.float32),
        scratch_types=[
            pltpu.VMEM((b_per_w,), jnp.int32),
            pltpu.VMEM((L, 2 * EMB), jnp.float32),
            pltpu.VMEM((L, 2 * EMB), jnp.float32),
            pltpu.VMEM((PAIRS, 2 * EMB), jnp.float32),
            pltpu.VMEM((PAIRS, 2 * EMB), jnp.float32),
            pltpu.SemaphoreType.DMA,
            pltpu.SemaphoreType.DMA,
            pltpu.SemaphoreType.DMA,
            pltpu.SemaphoreType.DMA,
        ],
        compiler_params=pltpu.CompilerParams(use_tc_tiling_on_sc=False),
    )
    def lookup(table_hbm, idx_hbm, mid_hbm,
               idx_v, rows0, rows1, obuf0, obuf1,
               gsem0, gsem1, ssem0, ssem1):
        rows = (rows0, rows1)
        obuf = (obuf0, obuf1)
        gsem = (gsem0, gsem1)
        ssem = (ssem0, ssem1)
        wid = lax.axis_index("s") * 2 + lax.axis_index("c")
        tbase = wid * b_per_w
        sbase = wid * (b_per_w // L)

        def gather(g, b):
            pltpu.async_copy(table_hbm.at[idx_v.at[pl.ds(g * L, L)]],
                             rows[b], gsem[b])

        def wait_gather(g, b):
            pltpu.make_async_copy(table_hbm.at[idx_v.at[pl.ds(g * L, L)]],
                                  rows[b], gsem[b]).wait()

        def scatter(g, b):
            pltpu.async_copy(obuf[b], mid_hbm.at[sbase + g], ssem[b])

        def wait_scatter(g, b):
            pltpu.make_async_copy(obuf[b], mid_hbm.at[sbase + g],
                                  ssem[b]).wait()

        def pack_scale(b):
            src = rows[b]
            dst = obuf[b]

            @plsc.parallel_loop(0, PAIRS, unroll=2)
            def _(i):
                for j in range(EMB // 16):
                    sl = pl.ds(j * 16, 16)
                    sh = pl.ds(EMB + j * 16, 16)
                    dst[i, sl] = src[2 * i, sl] * SCALE
                    dst[i, sh] = src[2 * i + 1, sl] * SCALE

        def step(g, b, first, last):
            other = 1 - b
            if not first:
                wait_scatter(g - 1, other)
            if not last:
                gather(g + 1, other)
            wait_gather(g, b)
            pack_scale(b)
            scatter(g, b)

        pltpu.sync_copy(idx_hbm.at[pl.ds(tbase, b_per_w)], idx_v)
        gather(0, 0)
        step(0, 0, first=True, last=False)
        step(1, 1, first=False, last=False)

        def outer(t, carry):
            g = 2 * t
            step(g, 0, first=False, last=False)
            step(g + 1, 1, first=False, last=False)
            return carry

        lax.fori_loop(1, n_chunks // 2 - 1, outer, 0)
        step(n_chunks - 2, 0, first=False, last=False)
        step(n_chunks - 1, 1, first=False, last=True)
        wait_scatter(n_chunks - 1, 1)

    return lookup


def kernel(token_sequences, table):
    Bseq, Lx = token_sequences.shape
    B = Bseq * Lx
    b_per_w = B // NUM_WORKERS
    n_chunks = b_per_w // L
    idx_flat = token_sequences.reshape(B)
    twide = _transpose_widen_table(table.T)
    mid = _make_sc_lookup(B, b_per_w, n_chunks)(twide, idx_flat)
    outT = _finish_transpose(mid, Bseq)
    return outT.transpose(2, 0, 1)


# speedup vs baseline: 2.1602x; 1.2367x over previous
import functools
import math

import jax
import jax.numpy as jnp
from jax import lax
from jax.experimental import pallas as pl
from jax.experimental.pallas import tpu as pltpu
from jax.experimental.pallas import tpu_sc as plsc

EMB = 64
SCALE = math.sqrt(EMB)

NUM_WORKERS = 32
L = 200
PAIRS = L // 2
PPAD = 104
TCOLS = 4096
CSEQ = 128


def _transpose_widen_table(tableT):
    D, V = tableT.shape

    def body(x_ref, o_ref):
        o_ref[:, :D] = x_ref[...].T

    return pl.pallas_call(
        body,
        grid=(pl.cdiv(V, TCOLS),),
        in_specs=[pl.BlockSpec((D, TCOLS), lambda i: (0, i))],
        out_specs=pl.BlockSpec((TCOLS, 2 * D), lambda i: (i, 0)),
        out_shape=jax.ShapeDtypeStruct((V, 2 * D), jnp.float32),
    )(tableT)


def _finish_transpose(mid, Bseq):
    def body(x_ref, o_ref):
        x = x_ref[...]
        for l in range(L):
            p, h = divmod(l, 2)
            o_ref[l] = x[:, p, h * EMB:(h + 1) * EMB].T

    return pl.pallas_call(
        body,
        grid=(Bseq // CSEQ,),
        in_specs=[pl.BlockSpec((CSEQ, PPAD, 2 * EMB), lambda i: (i, 0, 0))],
        out_specs=pl.BlockSpec((L, EMB, CSEQ), lambda i: (0, 0, i)),
        out_shape=jax.ShapeDtypeStruct((L, EMB, Bseq), jnp.float32),
    )(mid)


def _make_sc_lookup(B, b_per_w, n_chunks):
    mesh = plsc.VectorSubcoreMesh(core_axis_name="c", subcore_axis_name="s")

    @functools.partial(
        pl.kernel,
        mesh=mesh,
        out_type=jax.ShapeDtypeStruct((B // L, PPAD, 2 * EMB),
                                      jnp.float32),
        scratch_types=[
            pltpu.VMEM((b_per_w,), jnp.int32),
            pltpu.VMEM((L, 2 * EMB), jnp.float32),
            pltpu.VMEM((L, 2 * EMB), jnp.float32),
            pltpu.VMEM((PPAD, 2 * EMB), jnp.float32),
            pltpu.VMEM((PPAD, 2 * EMB), jnp.float32),
            pltpu.SemaphoreType.DMA,
            pltpu.SemaphoreType.DMA,
            pltpu.SemaphoreType.DMA,
            pltpu.SemaphoreType.DMA,
        ],
        compiler_params=pltpu.CompilerParams(use_tc_tiling_on_sc=False),
    )
    def lookup(table_hbm, idx_hbm, mid_hbm,
               idx_v, rows0, rows1, obuf0, obuf1,
               gsem0, gsem1, ssem0, ssem1):
        rows = (rows0, rows1)
        obuf = (obuf0, obuf1)
        gsem = (gsem0, gsem1)
        ssem = (ssem0, ssem1)
        wid = lax.axis_index("s") * 2 + lax.axis_index("c")
        tbase = wid * b_per_w
        sbase = wid * (b_per_w // L)

        def gather(g, b):
            pltpu.async_copy(table_hbm.at[idx_v.at[pl.ds(g * L, L)]],
                             rows[b], gsem[b])

        def wait_gather(g, b):
            pltpu.make_async_copy(table_hbm.at[idx_v.at[pl.ds(g * L, L)]],
                                  rows[b], gsem[b]).wait()

        def scatter(g, b):
            pltpu.async_copy(obuf[b], mid_hbm.at[sbase + g], ssem[b])

        def wait_scatter(g, b):
            pltpu.make_async_copy(obuf[b], mid_hbm.at[sbase + g],
                                  ssem[b]).wait()

        def pack_scale(b):
            src = rows[b]
            dst = obuf[b]

            @plsc.parallel_loop(0, PAIRS, unroll=2)
            def _(i):
                for j in range(EMB // 16):
                    sl = pl.ds(j * 16, 16)
                    sh = pl.ds(EMB + j * 16, 16)
                    dst[i, sl] = src[2 * i, sl] * SCALE
                    dst[i, sh] = src[2 * i + 1, sl] * SCALE

        def step(g, b, first, last):
            other = 1 - b
            if not first:
                wait_scatter(g - 1, other)
            if not last:
                gather(g + 1, other)
            wait_gather(g, b)
            pack_scale(b)
            scatter(g, b)

        pltpu.sync_copy(idx_hbm.at[pl.ds(tbase, b_per_w)], idx_v)
        gather(0, 0)
        step(0, 0, first=True, last=False)
        step(1, 1, first=False, last=False)

        def outer(t, carry):
            g = 2 * t
            step(g, 0, first=False, last=False)
            step(g + 1, 1, first=False, last=False)
            return carry

        lax.fori_loop(1, n_chunks // 2 - 1, outer, 0)
        step(n_chunks - 2, 0, first=False, last=False)
        step(n_chunks - 1, 1, first=False, last=True)
        wait_scatter(n_chunks - 1, 1)

    return lookup


def kernel(token_sequences, table):
    Bseq, Lx = token_sequences.shape
    B = Bseq * Lx
    b_per_w = B // NUM_WORKERS
    n_chunks = b_per_w // L
    idx_flat = token_sequences.reshape(B)
    twide = _transpose_widen_table(table.T)
    mid = _make_sc_lookup(B, b_per_w, n_chunks)(twide, idx_flat)
    outT = _finish_transpose(mid, Bseq)
    return outT.transpose(2, 0, 1)


# speedup vs baseline: 2.3785x; 1.1011x over previous
import functools
import math

import jax
import jax.numpy as jnp
from jax import lax
from jax.experimental import pallas as pl
from jax.experimental.pallas import tpu as pltpu
from jax.experimental.pallas import tpu_sc as plsc

EMB = 64
SCALE = math.sqrt(EMB)

NUM_WORKERS = 32
L = 200
PAIRS = L // 2
PPAD = 104
TCOLS = 8192
CSEQ = 128


def _transpose_widen_table(tableT):
    D, V = tableT.shape

    def body(x_ref, o_ref):
        o_ref[:, :D] = x_ref[...].T

    return pl.pallas_call(
        body,
        grid=(pl.cdiv(V, TCOLS),),
        in_specs=[pl.BlockSpec((D, TCOLS), lambda i: (0, i))],
        out_specs=pl.BlockSpec((TCOLS, 2 * D), lambda i: (i, 0)),
        out_shape=jax.ShapeDtypeStruct((V, 2 * D), jnp.float32),
    )(tableT)


def _finish_transpose(mid, Bseq):
    def body(x_ref, o_ref):
        x = x_ref[...]
        for l in range(L):
            p, h = divmod(l, 2)
            o_ref[l] = x[:, p, h * EMB:(h + 1) * EMB].T

    return pl.pallas_call(
        body,
        grid=(Bseq // CSEQ,),
        in_specs=[pl.BlockSpec((CSEQ, PPAD, 2 * EMB), lambda i: (i, 0, 0))],
        out_specs=pl.BlockSpec((L, EMB, CSEQ), lambda i: (0, 0, i)),
        out_shape=jax.ShapeDtypeStruct((L, EMB, Bseq), jnp.float32),
    )(mid)


def _make_sc_lookup(B, b_per_w, n_chunks):
    mesh = plsc.VectorSubcoreMesh(core_axis_name="c", subcore_axis_name="s")

    @functools.partial(
        pl.kernel,
        mesh=mesh,
        out_type=jax.ShapeDtypeStruct((B // L, PPAD, 2 * EMB),
                                      jnp.float32),
        scratch_types=[
            pltpu.VMEM((b_per_w,), jnp.int32),
            pltpu.VMEM((L, 2 * EMB), jnp.float32),
            pltpu.VMEM((L, 2 * EMB), jnp.float32),
            pltpu.VMEM((PPAD, 2 * EMB), jnp.float32),
            pltpu.VMEM((PPAD, 2 * EMB), jnp.float32),
            pltpu.SemaphoreType.DMA,
            pltpu.SemaphoreType.DMA,
            pltpu.SemaphoreType.DMA,
            pltpu.SemaphoreType.DMA,
        ],
        compiler_params=pltpu.CompilerParams(use_tc_tiling_on_sc=False),
    )
    def lookup(table_hbm, idx_hbm, mid_hbm,
               idx_v, rows0, rows1, obuf0, obuf1,
               gsem0, gsem1, ssem0, ssem1):
        rows = (rows0, rows1)
        obuf = (obuf0, obuf1)
        gsem = (gsem0, gsem1)
        ssem = (ssem0, ssem1)
        wid = lax.axis_index("s") * 2 + lax.axis_index("c")
        tbase = wid * b_per_w
        sbase = wid * (b_per_w // L)

        def gather(g, b):
            pltpu.async_copy(table_hbm.at[idx_v.at[pl.ds(g * L, L)]],
                             rows[b], gsem[b])

        def wait_gather(g, b):
            pltpu.make_async_copy(table_hbm.at[idx_v.at[pl.ds(g * L, L)]],
                                  rows[b], gsem[b]).wait()

        def scatter(g, b):
            pltpu.async_copy(obuf[b], mid_hbm.at[sbase + g], ssem[b])

        def wait_scatter(g, b):
            pltpu.make_async_copy(obuf[b], mid_hbm.at[sbase + g],
                                  ssem[b]).wait()

        def pack_scale(b):
            src = rows[b]
            dst = obuf[b]

            @plsc.parallel_loop(0, PAIRS, unroll=2)
            def _(i):
                for j in range(EMB // 16):
                    sl = pl.ds(j * 16, 16)
                    sh = pl.ds(EMB + j * 16, 16)
                    dst[i, sl] = src[2 * i, sl] * SCALE
                    dst[i, sh] = src[2 * i + 1, sl] * SCALE

        def step(g, b, first, last):
            other = 1 - b
            if not first:
                wait_scatter(g - 1, other)
            if not last:
                gather(g + 1, other)
            wait_gather(g, b)
            pack_scale(b)
            scatter(g, b)

        pltpu.sync_copy(idx_hbm.at[pl.ds(tbase, b_per_w)], idx_v)
        gather(0, 0)
        step(0, 0, first=True, last=False)
        step(1, 1, first=False, last=False)

        def outer(t, carry):
            g = 2 * t
            step(g, 0, first=False, last=False)
            step(g + 1, 1, first=False, last=False)
            return carry

        lax.fori_loop(1, n_chunks // 2 - 1, outer, 0)
        step(n_chunks - 2, 0, first=False, last=False)
        step(n_chunks - 1, 1, first=False, last=True)
        wait_scatter(n_chunks - 1, 1)

    return lookup


def kernel(token_sequences, table):
    Bseq, Lx = token_sequences.shape
    B = Bseq * Lx
    b_per_w = B // NUM_WORKERS
    n_chunks = b_per_w // L
    idx_flat = token_sequences.reshape(B)
    twide = _transpose_widen_table(table.T)
    mid = _make_sc_lookup(B, b_per_w, n_chunks)(twide, idx_flat)
    outT = _finish_transpose(mid, Bseq)
    return outT.transpose(2, 0, 1)


# speedup vs baseline: 2.4440x; 1.0275x over previous
import functools
import math

import jax
import jax.numpy as jnp
from jax import lax
from jax.experimental import pallas as pl
from jax.experimental.pallas import tpu as pltpu
from jax.experimental.pallas import tpu_sc as plsc

EMB = 64
SCALE = math.sqrt(EMB)

NUM_WORKERS = 32
L = 200
PAIRS = L // 2
PPAD = 104
TCOLS = 16384
CSEQ = 128


def _transpose_widen_table(tableT):
    D, V = tableT.shape

    def body(x_ref, o_ref):
        o_ref[:, :D] = x_ref[...].T

    return pl.pallas_call(
        body,
        grid=(pl.cdiv(V, TCOLS),),
        in_specs=[pl.BlockSpec((D, TCOLS), lambda i: (0, i))],
        out_specs=pl.BlockSpec((TCOLS, 2 * D), lambda i: (i, 0)),
        out_shape=jax.ShapeDtypeStruct((V, 2 * D), jnp.float32),
    )(tableT)


def _finish_transpose(mid, Bseq):
    def body(x_ref, o_ref):
        x = x_ref[...]
        for l in range(L):
            p, h = divmod(l, 2)
            o_ref[l] = x[:, p, h * EMB:(h + 1) * EMB].T

    return pl.pallas_call(
        body,
        grid=(Bseq // CSEQ,),
        in_specs=[pl.BlockSpec((CSEQ, PPAD, 2 * EMB), lambda i: (i, 0, 0))],
        out_specs=pl.BlockSpec((L, EMB, CSEQ), lambda i: (0, 0, i)),
        out_shape=jax.ShapeDtypeStruct((L, EMB, Bseq), jnp.float32),
    )(mid)


def _make_sc_lookup(B, b_per_w, n_chunks):
    mesh = plsc.VectorSubcoreMesh(core_axis_name="c", subcore_axis_name="s")

    @functools.partial(
        pl.kernel,
        mesh=mesh,
        out_type=jax.ShapeDtypeStruct((B // L, PPAD, 2 * EMB),
                                      jnp.float32),
        scratch_types=[
            pltpu.VMEM((b_per_w,), jnp.int32),
            pltpu.VMEM((L, 2 * EMB), jnp.float32),
            pltpu.VMEM((L, 2 * EMB), jnp.float32),
            pltpu.VMEM((PPAD, 2 * EMB), jnp.float32),
            pltpu.VMEM((PPAD, 2 * EMB), jnp.float32),
            pltpu.SemaphoreType.DMA,
            pltpu.SemaphoreType.DMA,
            pltpu.SemaphoreType.DMA,
            pltpu.SemaphoreType.DMA,
        ],
        compiler_params=pltpu.CompilerParams(use_tc_tiling_on_sc=False),
    )
    def lookup(table_hbm, idx_hbm, mid_hbm,
               idx_v, rows0, rows1, obuf0, obuf1,
               gsem0, gsem1, ssem0, ssem1):
        rows = (rows0, rows1)
        obuf = (obuf0, obuf1)
        gsem = (gsem0, gsem1)
        ssem = (ssem0, ssem1)
        wid = lax.axis_index("s") * 2 + lax.axis_index("c")
        tbase = wid * b_per_w
        sbase = wid * (b_per_w // L)

        def gather(g, b):
            pltpu.async_copy(table_hbm.at[idx_v.at[pl.ds(g * L, L)]],
                             rows[b], gsem[b])

        def wait_gather(g, b):
            pltpu.make_async_copy(table_hbm.at[idx_v.at[pl.ds(g * L, L)]],
                                  rows[b], gsem[b]).wait()

        def scatter(g, b):
            pltpu.async_copy(obuf[b], mid_hbm.at[sbase + g], ssem[b])

        def wait_scatter(g, b):
            pltpu.make_async_copy(obuf[b], mid_hbm.at[sbase + g],
                                  ssem[b]).wait()

        def pack_scale(b):
            src = rows[b]
            dst = obuf[b]

            @plsc.parallel_loop(0, PAIRS, unroll=2)
            def _(i):
                for j in range(EMB // 16):
                    sl = pl.ds(j * 16, 16)
                    sh = pl.ds(EMB + j * 16, 16)
                    dst[i, sl] = src[2 * i, sl] * SCALE
                    dst[i, sh] = src[2 * i + 1, sl] * SCALE

        def step(g, b, first, last):
            other = 1 - b
            if not first:
                wait_scatter(g - 1, other)
            if not last:
                gather(g + 1, other)
            wait_gather(g, b)
            pack_scale(b)
            scatter(g, b)

        pltpu.sync_copy(idx_hbm.at[pl.ds(tbase, b_per_w)], idx_v)
        gather(0, 0)
        step(0, 0, first=True, last=False)
        step(1, 1, first=False, last=False)

        def outer(t, carry):
            g = 2 * t
            step(g, 0, first=False, last=False)
            step(g + 1, 1, first=False, last=False)
            return carry

        lax.fori_loop(1, n_chunks // 2 - 1, outer, 0)
        step(n_chunks - 2, 0, first=False, last=False)
        step(n_chunks - 1, 1, first=False, last=True)
        wait_scatter(n_chunks - 1, 1)

    return lookup


def kernel(token_sequences, table):
    Bseq, Lx = token_sequences.shape
    B = Bseq * Lx
    b_per_w = B // NUM_WORKERS
    n_chunks = b_per_w // L
    idx_flat = token_sequences.reshape(B)
    twide = _transpose_widen_table(table.T)
    mid = _make_sc_lookup(B, b_per_w, n_chunks)(twide, idx_flat)
    outT = _finish_transpose(mid, Bseq)
    return outT.transpose(2, 0, 1)


# speedup vs baseline: 2.4698x; 1.0106x over previous
import functools
import math

import jax
import jax.numpy as jnp
from jax import lax
from jax.experimental import pallas as pl
from jax.experimental.pallas import tpu as pltpu
from jax.experimental.pallas import tpu_sc as plsc

EMB = 64
SCALE = math.sqrt(EMB)

NUM_WORKERS = 32
L = 200
PAIRS = L // 2
PPAD = 104
TCOLS = 32768
CSEQ = 128


def _transpose_widen_table(tableT):
    D, V = tableT.shape

    def body(x_ref, o_ref):
        o_ref[:, :D] = x_ref[...].T

    return pl.pallas_call(
        body,
        grid=(pl.cdiv(V, TCOLS),),
        in_specs=[pl.BlockSpec((D, TCOLS), lambda i: (0, i))],
        out_specs=pl.BlockSpec((TCOLS, 2 * D), lambda i: (i, 0)),
        out_shape=jax.ShapeDtypeStruct((V, 2 * D), jnp.float32),
    )(tableT)


def _finish_transpose(mid, Bseq):
    def body(x_ref, o_ref):
        x = x_ref[...]
        for l in range(L):
            p, h = divmod(l, 2)
            o_ref[l] = x[:, p, h * EMB:(h + 1) * EMB].T

    return pl.pallas_call(
        body,
        grid=(Bseq // CSEQ,),
        in_specs=[pl.BlockSpec((CSEQ, PPAD, 2 * EMB), lambda i: (i, 0, 0))],
        out_specs=pl.BlockSpec((L, EMB, CSEQ), lambda i: (0, 0, i)),
        out_shape=jax.ShapeDtypeStruct((L, EMB, Bseq), jnp.float32),
    )(mid)


def _make_sc_lookup(B, b_per_w, n_chunks):
    mesh = plsc.VectorSubcoreMesh(core_axis_name="c", subcore_axis_name="s")

    @functools.partial(
        pl.kernel,
        mesh=mesh,
        out_type=jax.ShapeDtypeStruct((B // L, PPAD, 2 * EMB),
                                      jnp.float32),
        scratch_types=[
            pltpu.VMEM((b_per_w,), jnp.int32),
            pltpu.VMEM((L, 2 * EMB), jnp.float32),
            pltpu.VMEM((L, 2 * EMB), jnp.float32),
            pltpu.VMEM((PPAD, 2 * EMB), jnp.float32),
            pltpu.VMEM((PPAD, 2 * EMB), jnp.float32),
            pltpu.SemaphoreType.DMA,
            pltpu.SemaphoreType.DMA,
            pltpu.SemaphoreType.DMA,
            pltpu.SemaphoreType.DMA,
        ],
        compiler_params=pltpu.CompilerParams(use_tc_tiling_on_sc=False),
    )
    def lookup(table_hbm, idx_hbm, mid_hbm,
               idx_v, rows0, rows1, obuf0, obuf1,
               gsem0, gsem1, ssem0, ssem1):
        rows = (rows0, rows1)
        obuf = (obuf0, obuf1)
        gsem = (gsem0, gsem1)
        ssem = (ssem0, ssem1)
        wid = lax.axis_index("s") * 2 + lax.axis_index("c")
        tbase = wid * b_per_w
        sbase = wid * (b_per_w // L)

        def gather(g, b):
            pltpu.async_copy(table_hbm.at[idx_v.at[pl.ds(g * L, L)]],
                             rows[b], gsem[b])

        def wait_gather(g, b):
            pltpu.make_async_copy(table_hbm.at[idx_v.at[pl.ds(g * L, L)]],
                                  rows[b], gsem[b]).wait()

        def scatter(g, b):
            pltpu.async_copy(obuf[b], mid_hbm.at[sbase + g], ssem[b])

        def wait_scatter(g, b):
            pltpu.make_async_copy(obuf[b], mid_hbm.at[sbase + g],
                                  ssem[b]).wait()

        def pack_scale(b):
            src = rows[b]
            dst = obuf[b]

            @plsc.parallel_loop(0, PAIRS, unroll=2)
            def _(i):
                for j in range(EMB // 16):
                    sl = pl.ds(j * 16, 16)
                    sh = pl.ds(EMB + j * 16, 16)
                    dst[i, sl] = src[2 * i, sl] * SCALE
                    dst[i, sh] = src[2 * i + 1, sl] * SCALE

        def step(g, b, first, last):
            other = 1 - b
            if not first:
                wait_scatter(g - 1, other)
            if not last:
                gather(g + 1, other)
            wait_gather(g, b)
            pack_scale(b)
            scatter(g, b)

        pltpu.sync_copy(idx_hbm.at[pl.ds(tbase, b_per_w)], idx_v)
        gather(0, 0)
        step(0, 0, first=True, last=False)
        step(1, 1, first=False, last=False)

        def outer(t, carry):
            g = 2 * t
            step(g, 0, first=False, last=False)
            step(g + 1, 1, first=False, last=False)
            return carry

        lax.fori_loop(1, n_chunks // 2 - 1, outer, 0)
        step(n_chunks - 2, 0, first=False, last=False)
        step(n_chunks - 1, 1, first=False, last=True)
        wait_scatter(n_chunks - 1, 1)

    return lookup


def kernel(token_sequences, table):
    Bseq, Lx = token_sequences.shape
    B = Bseq * Lx
    b_per_w = B // NUM_WORKERS
    n_chunks = b_per_w // L
    idx_flat = token_sequences.reshape(B)
    twide = _transpose_widen_table(table.T)
    mid = _make_sc_lookup(B, b_per_w, n_chunks)(twide, idx_flat)
    outT = _finish_transpose(mid, Bseq)
    return outT.transpose(2, 0, 1)
